# Initial kernel scaffold; baseline (speedup 1.0000x reference)
import jax
import jax.numpy as jnp
from jax.experimental import pallas as pl


def kernel(scores, p_trade, y_rank, y_trade, weights, mask):
    raise NotImplementedError("write your pallas kernel here")


# speedup vs baseline: 6.0263x; 6.0263x over previous
import jax
import jax.numpy as jnp
from jax.experimental import pallas as pl
from jax.experimental.pallas import tpu as pltpu

_TRADE_LAMBDA = 0.25
_B = 64
_N = 1024
_K = 204
_KPAD = 256


def _loss_kernel(s_ref, p_ref, yr_ref, yt_ref, w_ref, m_ref, out_ref, acc_ref):
    b = pl.program_id(0)

    @pl.when(b == 0)
    def _init():
        for i in range(6):
            acc_ref[i] = 0.0

    y = yr_ref[0, 0, :]
    s = s_ref[0, 0, :]
    w = w_ref[0, 0, :]

    ycol = y[:, None]
    yrow = y[None, :]
    jlt = jax.lax.broadcasted_iota(jnp.int32, (_N, _N), 1) < \
        jax.lax.broadcasted_iota(jnp.int32, (_N, _N), 0)
    ge = (yrow >= ycol).astype(jnp.int32)
    gt = (yrow > ycol).astype(jnp.int32)
    rank = jnp.sum(jnp.where(jlt, ge, gt), axis=1)

    r_iota = jax.lax.broadcasted_iota(jnp.int32, (_KPAD, _N), 0)
    rrow = rank[None, :]
    sel_top = ((rrow == r_iota) & (r_iota < _K)).astype(jnp.float32)
    sel_bot = (rrow == r_iota + (_N - _K)).astype(jnp.float32)

    x = jnp.concatenate([s[:, None], w[:, None]], axis=1)
    top = jnp.dot(sel_top, x, preferred_element_type=jnp.float32)
    bot = jnp.dot(sel_bot, x, preferred_element_type=jnp.float32)
    st = top[:, 0]
    at = jnp.sqrt(top[:, 1])
    sb = bot[:, 0]
    ab = jnp.sqrt(bot[:, 1])

    d = sb[None, :] - st[:, None]
    sp = jnp.maximum(d, 0.0) + jnp.log1p(jnp.exp(-jnp.abs(d)))
    wp = at[:, None] * ab[None, :]
    num = jnp.sum(sp * wp)
    den = jnp.sum(at) * jnp.sum(ab)
    l_rank = num / (den + 1e-8)

    p = p_ref[0, 0, :]
    ytr = yt_ref[0, 0, :]
    m = m_ref[0, 0, :]
    logp = jnp.maximum(jnp.log(p), -100.0)
    log1mp = jnp.maximum(jnp.log(1.0 - p), -100.0)
    bce = -(ytr * logp + (1.0 - ytr) * log1mp)
    mw = w * m
    t_den = jnp.sum(mw)
    t_num = jnp.sum(bce * mw)
    pb_trade = t_num / (t_den + 1e-8)
    valid = t_den > 0.0

    acc_ref[0] += l_rank
    acc_ref[1] += jnp.where(valid, pb_trade, 0.0)
    acc_ref[2] += jnp.where(valid, 1.0, 0.0)
    acc_ref[3] += jnp.sum(p * m)
    acc_ref[4] += jnp.sum(m)

    @pl.when(b == _B - 1)
    def _finish():
        avg_rank = acc_ref[0] / float(_B)
        avg_trade = acc_ref[1] / jnp.maximum(acc_ref[2], 1.0)
        out_ref[0] = avg_rank + _TRADE_LAMBDA * avg_trade
        out_ref[1] = avg_rank
        out_ref[2] = avg_trade
        out_ref[3] = acc_ref[3] / jnp.maximum(acc_ref[4], 1.0)


def kernel(scores, p_trade, y_rank, y_trade, weights, mask):
    row = pl.BlockSpec((1, 1, _N), lambda b: (b, 0, 0))
    args = (scores, p_trade, y_rank, y_trade, weights,
            mask.astype(jnp.float32))
    args = tuple(a.reshape(_B, 1, _N) for a in args)
    out = pl.pallas_call(
        _loss_kernel,
        grid=(_B,),
        in_specs=[row] * 6,
        out_specs=pl.BlockSpec(memory_space=pltpu.SMEM),
        out_shape=jax.ShapeDtypeStruct((4,), jnp.float32),
        scratch_shapes=[pltpu.SMEM((6,), jnp.float32)],
    )(*args)
    return (out[0], out[1], out[2], out[3])


# speedup vs baseline: 22.7900x; 3.7818x over previous
import jax
import jax.numpy as jnp
from jax.experimental import pallas as pl
from jax.experimental.pallas import tpu as pltpu

_TRADE_LAMBDA = 0.25
_B = 64
_N = 1024
_K = 204
_KPAD = 256
_MINT = -(1 << 31)


def _cumsum_lanes(x):
    r, c = x.shape
    sh = 1
    while sh < c:
        z = jnp.zeros((r, sh), x.dtype)
        x = x + jnp.concatenate([z, x[:, :c - sh]], axis=1)
        sh *= 2
    return x


def _select_kernel(y_ref, out_ref):
    y = y_ref[...]
    y = jnp.where(y == 0.0, 0.0, y)
    bits = jax.lax.bitcast_convert_type(y, jnp.int32)
    key = jnp.where(bits >= 0, bits, ~(bits & 0x7FFFFFFF))
    k2 = jnp.concatenate([key, ~key], axis=0)

    tu = jnp.zeros((2 * _B, 1), jnp.int32)
    for bit in range(31, -1, -1):
        bv = jnp.asarray(_MINT if bit == 31 else (1 << bit), jnp.int32)
        cand = tu | bv
        scand = cand ^ _MINT
        cnt = jnp.sum((k2 >= scand).astype(jnp.int32), axis=1, keepdims=True)
        tu = jnp.where(cnt >= _K, cand, tu)
    thr = tu ^ _MINT

    gti = (k2 > thr).astype(jnp.int32)
    eqi = (k2 == thr).astype(jnp.int32)
    g = jnp.sum(gti, axis=1, keepdims=True)
    need = _K - g

    incl = _cumsum_lanes(eqi)
    tot = jnp.sum(eqi, axis=1, keepdims=True)
    pos_l = incl - eqi
    pos_r = tot - incl
    rowi = jax.lax.broadcasted_iota(jnp.int32, (2 * _B, _N), 0)
    tie_pos = jnp.where(rowi < _B, pos_l, pos_r)
    sel = gti + eqi * (tie_pos < need).astype(jnp.int32)

    slot = _cumsum_lanes(sel) - sel
    out_ref[...] = jnp.where(sel > 0, slot, -1)


def _loss_kernel(s_ref, p_ref, yt_ref, w_ref, m_ref, pt_ref, pb_ref,
                 out_ref, acc_ref):
    b = pl.program_id(0)

    @pl.when(b == 0)
    def _init():
        for i in range(6):
            acc_ref[i] = 0.0

    s = s_ref[0, 0, :]
    w = w_ref[0, 0, :]
    ptop = pt_ref[0, 0, :]
    pbot = pb_ref[0, 0, :]

    r256 = jax.lax.broadcasted_iota(jnp.int32, (_KPAD, _N), 0)
    p_top = (ptop[None, :] == r256).astype(jnp.float32)
    p_bot = (pbot[None, :] == r256).astype(jnp.float32)
    sel = jnp.concatenate([p_top, p_bot], axis=0)
    x = jnp.concatenate([s[:, None], w[:, None]], axis=1)
    gath = jnp.dot(sel, x, preferred_element_type=jnp.float32)
    st = gath[:_KPAD, 0]
    at = jnp.sqrt(gath[:_KPAD, 1])
    sb = gath[_KPAD:, 0]
    ab = jnp.sqrt(gath[_KPAD:, 1])

    d = sb[None, :] - st[:, None]
    sp = jnp.maximum(d, 0.0) + jnp.log1p(jnp.exp(-jnp.abs(d)))
    wp = at[:, None] * ab[None, :]
    num = jnp.sum(sp * wp)
    den = jnp.sum(at) * jnp.sum(ab)
    l_rank = num / (den + 1e-8)

    p = p_ref[0, 0, :]
    ytr = yt_ref[0, 0, :]
    m = m_ref[0, 0, :]
    logp = jnp.maximum(jnp.log(p), -100.0)
    log1mp = jnp.maximum(jnp.log(1.0 - p), -100.0)
    bce = -(ytr * logp + (1.0 - ytr) * log1mp)
    mw = w * m
    t_den = jnp.sum(mw)
    t_num = jnp.sum(bce * mw)
    pb_trade = t_num / (t_den + 1e-8)
    valid = t_den > 0.0

    acc_ref[0] += l_rank
    acc_ref[1] += jnp.where(valid, pb_trade, 0.0)
    acc_ref[2] += jnp.where(valid, 1.0, 0.0)
    acc_ref[3] += jnp.sum(p * m)
    acc_ref[4] += jnp.sum(m)

    @pl.when(b == _B - 1)
    def _finish():
        avg_rank = acc_ref[0] / float(_B)
        avg_trade = acc_ref[1] / jnp.maximum(acc_ref[2], 1.0)
        out_ref[0] = avg_rank + _TRADE_LAMBDA * avg_trade
        out_ref[1] = avg_rank
        out_ref[2] = avg_trade
        out_ref[3] = acc_ref[3] / jnp.maximum(acc_ref[4], 1.0)


def kernel(scores, p_trade, y_rank, y_trade, weights, mask):
    slots = pl.pallas_call(
        _select_kernel,
        out_shape=jax.ShapeDtypeStruct((2 * _B, _N), jnp.int32),
    )(y_rank)
    ptop = slots[:_B].reshape(_B, 1, _N)
    pbot = slots[_B:].reshape(_B, 1, _N)

    row = pl.BlockSpec((1, 1, _N), lambda b: (b, 0, 0))
    args = (scores, p_trade, y_trade, weights, mask.astype(jnp.float32))
    args = tuple(a.reshape(_B, 1, _N) for a in args) + (ptop, pbot)
    out = pl.pallas_call(
        _loss_kernel,
        grid=(_B,),
        in_specs=[row] * 7,
        out_specs=pl.BlockSpec(memory_space=pltpu.SMEM),
        out_shape=jax.ShapeDtypeStruct((4,), jnp.float32),
        scratch_shapes=[pltpu.SMEM((6,), jnp.float32)],
    )(*args)
    return (out[0], out[1], out[2], out[3])


# speedup vs baseline: 524.2898x; 23.0053x over previous
import jax
import jax.numpy as jnp
from jax.experimental import pallas as pl
from jax.experimental.pallas import tpu as pltpu

_TRADE_LAMBDA = 0.25
_B = 64
_N = 1024
_K = 204
_KPAD = 256
_MINT = -(1 << 31)


def _cumsum_lanes(x):
    r, c = x.shape
    sh = 1
    while sh < c:
        z = jnp.zeros((r, sh), x.dtype)
        x = x + jnp.concatenate([z, x[:, :c - sh]], axis=1)
        sh *= 2
    return x


def _select_kernel(y_ref, out_ref):
    y = y_ref[...]
    y = jnp.where(y == 0.0, 0.0, y)
    bits = jax.lax.bitcast_convert_type(y, jnp.int32)
    key = jnp.where(bits >= 0, bits, ~(bits & 0x7FFFFFFF))
    k2 = jnp.concatenate([key, ~key], axis=0)

    tu = jnp.zeros((2 * _B, 1), jnp.int32)
    for bit in range(31, -1, -1):
        bv = jnp.asarray(_MINT if bit == 31 else (1 << bit), jnp.int32)
        cand = tu | bv
        scand = cand ^ _MINT
        cnt = jnp.sum((k2 >= scand).astype(jnp.int32), axis=1, keepdims=True)
        tu = jnp.where(cnt >= _K, cand, tu)
    thr = tu ^ _MINT

    gti = (k2 > thr).astype(jnp.int32)
    eqi = (k2 == thr).astype(jnp.int32)
    g = jnp.sum(gti, axis=1, keepdims=True)
    need = _K - g

    incl = _cumsum_lanes(eqi)
    tot = jnp.sum(eqi, axis=1, keepdims=True)
    pos_l = incl - eqi
    pos_r = tot - incl
    rowi = jax.lax.broadcasted_iota(jnp.int32, (2 * _B, _N), 0)
    tie_pos = jnp.where(rowi < _B, pos_l, pos_r)
    sel = gti + eqi * (tie_pos < need).astype(jnp.int32)

    slot = _cumsum_lanes(sel) - sel
    out_ref[...] = jnp.where(sel > 0, slot, -1)


def _loss_kernel(s_ref, p_ref, yt_ref, w_ref, m_ref, pt_ref, pb_ref,
                 out_ref, acc_ref):
    b = pl.program_id(0)

    @pl.when(b == 0)
    def _init():
        for i in range(6):
            acc_ref[i] = 0.0

    sw = jnp.concatenate([s_ref[0], w_ref[0]], axis=0)
    ptop = pt_ref[0]
    pbot = pb_ref[0]

    r256 = jax.lax.broadcasted_iota(jnp.int32, (_KPAD, _N), 0)
    p_top = (ptop == r256).astype(jnp.float32)
    p_bot = (pbot == r256).astype(jnp.float32)
    dnum_c = (((1,), (1,)), ((), ()))
    top_g = jax.lax.dot_general(p_top, sw, dnum_c,
                                preferred_element_type=jnp.float32)
    bot_g = jax.lax.dot_general(sw, p_bot, dnum_c,
                                preferred_element_type=jnp.float32)
    st = top_g[:, 0:1]
    at = jnp.sqrt(top_g[:, 1:2])
    sb = bot_g[0:1, :]
    ab = jnp.sqrt(bot_g[1:2, :])

    d = sb - st
    sp = jnp.maximum(d, 0.0) + jnp.log1p(jnp.exp(-jnp.abs(d)))
    wp = at * ab
    num = jnp.sum(sp * wp)
    den = jnp.sum(at) * jnp.sum(ab)
    l_rank = num / (den + 1e-8)

    w = w_ref[0, 0, :]
    p = p_ref[0, 0, :]
    ytr = yt_ref[0, 0, :]
    m = m_ref[0, 0, :]
    logp = jnp.maximum(jnp.log(p), -100.0)
    log1mp = jnp.maximum(jnp.log(1.0 - p), -100.0)
    bce = -(ytr * logp + (1.0 - ytr) * log1mp)
    mw = w * m
    t_den = jnp.sum(mw)
    t_num = jnp.sum(bce * mw)
    pb_trade = t_num / (t_den + 1e-8)
    valid = t_den > 0.0

    acc_ref[0] += l_rank
    acc_ref[1] += jnp.where(valid, pb_trade, 0.0)
    acc_ref[2] += jnp.where(valid, 1.0, 0.0)
    acc_ref[3] += jnp.sum(p * m)
    acc_ref[4] += jnp.sum(m)

    @pl.when(b == _B - 1)
    def _finish():
        avg_rank = acc_ref[0] / float(_B)
        avg_trade = acc_ref[1] / jnp.maximum(acc_ref[2], 1.0)
        out_ref[0] = avg_rank + _TRADE_LAMBDA * avg_trade
        out_ref[1] = avg_rank
        out_ref[2] = avg_trade
        out_ref[3] = acc_ref[3] / jnp.maximum(acc_ref[4], 1.0)


def kernel(scores, p_trade, y_rank, y_trade, weights, mask):
    slots = pl.pallas_call(
        _select_kernel,
        out_shape=jax.ShapeDtypeStruct((2 * _B, _N), jnp.int32),
    )(y_rank)
    ptop = slots[:_B].reshape(_B, 1, _N)
    pbot = slots[_B:].reshape(_B, 1, _N)

    row = pl.BlockSpec((1, 1, _N), lambda b: (b, 0, 0))
    args = (scores, p_trade, y_trade, weights, mask.astype(jnp.float32))
    args = tuple(a.reshape(_B, 1, _N) for a in args) + (ptop, pbot)
    out = pl.pallas_call(
        _loss_kernel,
        grid=(_B,),
        in_specs=[row] * 7,
        out_specs=pl.BlockSpec(memory_space=pltpu.SMEM),
        out_shape=jax.ShapeDtypeStruct((4,), jnp.float32),
        scratch_shapes=[pltpu.SMEM((6,), jnp.float32)],
    )(*args)
    return (out[0], out[1], out[2], out[3])


# speedup vs baseline: 668.2577x; 1.2746x over previous
import jax
import jax.numpy as jnp
from jax.experimental import pallas as pl
from jax.experimental.pallas import tpu as pltpu

_TRADE_LAMBDA = 0.25
_B = 64
_N = 1024
_K = 204
_KPAD = 256
_MINT = -(1 << 31)
_BPS = 2


def _cumsum_lanes(x):
    r, c = x.shape
    sh = 1
    while sh < c:
        z = jnp.zeros((r, sh), x.dtype)
        x = x + jnp.concatenate([z, x[:, :c - sh]], axis=1)
        sh *= 2
    return x


def _select_kernel(y_ref, out_ref):
    y = y_ref[...]
    y = jnp.where(y == 0.0, 0.0, y)
    bits = jax.lax.bitcast_convert_type(y, jnp.int32)
    key = jnp.where(bits >= 0, bits, ~(bits & 0x7FFFFFFF))
    k2 = jnp.concatenate([key, ~key], axis=0)

    tu = jnp.zeros((2 * _B, 1), jnp.int32)
    for bit in range(31, -1, -1):
        bv = jnp.asarray(_MINT if bit == 31 else (1 << bit), jnp.int32)
        cand = tu | bv
        scand = cand ^ _MINT
        cnt = jnp.sum((k2 >= scand).astype(jnp.int32), axis=1, keepdims=True)
        tu = jnp.where(cnt >= _K, cand, tu)
    thr = tu ^ _MINT

    gti = (k2 > thr).astype(jnp.int32)
    eqi = (k2 == thr).astype(jnp.int32)
    g = jnp.sum(gti, axis=1, keepdims=True)
    need = _K - g

    incl = _cumsum_lanes(eqi)
    tot = jnp.sum(eqi, axis=1, keepdims=True)
    pos_l = incl - eqi
    pos_r = tot - incl
    rowi = jax.lax.broadcasted_iota(jnp.int32, (2 * _B, _N), 0)
    tie_pos = jnp.where(rowi < _B, pos_l, pos_r)
    sel = gti + eqi * (tie_pos < need).astype(jnp.int32)

    slot = _cumsum_lanes(sel) - sel
    out_ref[...] = jnp.where(sel > 0, slot, -1)


def _loss_kernel(s_ref, p_ref, yt_ref, w_ref, m_ref, pt_ref, pb_ref,
                 out_ref, acc_ref):
    b = pl.program_id(0)

    @pl.when(b == 0)
    def _init():
        for i in range(6):
            acc_ref[i] = 0.0

    r256 = jax.lax.broadcasted_iota(jnp.int32, (_KPAD, _N), 0)
    dnum_c = (((1,), (1,)), ((), ()))
    rank_part = 0.0
    for t in range(_BPS):
        sw = jnp.concatenate([s_ref[t], w_ref[t]], axis=0)
        p_top = (pt_ref[t] == r256).astype(jnp.float32)
        p_bot = (pb_ref[t] == r256).astype(jnp.float32)
        top_g = jax.lax.dot_general(p_top, sw, dnum_c,
                                    preferred_element_type=jnp.float32)
        bot_g = jax.lax.dot_general(sw, p_bot, dnum_c,
                                    preferred_element_type=jnp.float32)
        st = top_g[:, 0:1]
        at = jnp.sqrt(top_g[:, 1:2])
        sb = bot_g[0:1, :]
        ab = jnp.sqrt(bot_g[1:2, :])

        d = sb - st
        sp = jnp.maximum(d, 0.0) + jnp.log1p(jnp.exp(-jnp.abs(d)))
        wp = at * ab
        num = jnp.sum(sp * wp)
        den = jnp.sum(at) * jnp.sum(ab)
        rank_part += num / (den + 1e-8)

    w = w_ref[:, 0, :]
    p = p_ref[:, 0, :]
    ytr = yt_ref[:, 0, :]
    m = m_ref[:, 0, :]
    logp = jnp.maximum(jnp.log(p), -100.0)
    log1mp = jnp.maximum(jnp.log(1.0 - p), -100.0)
    bce = -(ytr * logp + (1.0 - ytr) * log1mp)
    mw = w * m
    t_den = jnp.sum(mw, axis=1, keepdims=True)
    t_num = jnp.sum(bce * mw, axis=1, keepdims=True)
    validf = (t_den > 0.0).astype(jnp.float32)
    pb_trade = t_num / (t_den + 1e-8)

    acc_ref[0] += rank_part
    acc_ref[1] += jnp.sum(validf * pb_trade)
    acc_ref[2] += jnp.sum(validf)
    acc_ref[3] += jnp.sum(p * m)
    acc_ref[4] += jnp.sum(m)

    @pl.when(b == _B // _BPS - 1)
    def _finish():
        avg_rank = acc_ref[0] / float(_B)
        avg_trade = acc_ref[1] / jnp.maximum(acc_ref[2], 1.0)
        out_ref[0] = avg_rank + _TRADE_LAMBDA * avg_trade
        out_ref[1] = avg_rank
        out_ref[2] = avg_trade
        out_ref[3] = acc_ref[3] / jnp.maximum(acc_ref[4], 1.0)


def kernel(scores, p_trade, y_rank, y_trade, weights, mask):
    slots = pl.pallas_call(
        _select_kernel,
        out_shape=jax.ShapeDtypeStruct((2 * _B, _N), jnp.int32),
    )(y_rank)
    ptop = slots[:_B].reshape(_B, 1, _N)
    pbot = slots[_B:].reshape(_B, 1, _N)

    blk = pl.BlockSpec((_BPS, 1, _N), lambda b: (b, 0, 0))
    args = (scores, p_trade, y_trade, weights, mask.astype(jnp.float32))
    args = tuple(a.reshape(_B, 1, _N) for a in args) + (ptop, pbot)
    out = pl.pallas_call(
        _loss_kernel,
        grid=(_B // _BPS,),
        in_specs=[blk] * 7,
        out_specs=pl.BlockSpec(memory_space=pltpu.SMEM),
        out_shape=jax.ShapeDtypeStruct((4,), jnp.float32),
        scratch_shapes=[pltpu.SMEM((6,), jnp.float32)],
    )(*args)
    return (out[0], out[1], out[2], out[3])


# speedup vs baseline: 752.9588x; 1.1267x over previous
import jax
import jax.numpy as jnp
from jax.experimental import pallas as pl
from jax.experimental.pallas import tpu as pltpu

_TRADE_LAMBDA = 0.25
_B = 64
_N = 1024
_K = 204
_KPAD = 256
_MINT = -(1 << 31)
_BPS = 4


def _cumsum_lanes(x):
    r, c = x.shape
    sh = 1
    while sh < c:
        z = jnp.zeros((r, sh), x.dtype)
        x = x + jnp.concatenate([z, x[:, :c - sh]], axis=1)
        sh *= 2
    return x


def _select_kernel(y_ref, out_ref):
    y = y_ref[...]
    y = jnp.where(y == 0.0, 0.0, y)
    bits = jax.lax.bitcast_convert_type(y, jnp.int32)
    key = jnp.where(bits >= 0, bits, ~(bits & 0x7FFFFFFF))
    k2 = jnp.concatenate([key, ~key], axis=0)

    tu = jnp.zeros((2 * _B, 1), jnp.int32)
    for bit in range(31, -1, -1):
        bv = jnp.asarray(_MINT if bit == 31 else (1 << bit), jnp.int32)
        cand = tu | bv
        scand = cand ^ _MINT
        cnt = jnp.sum((k2 >= scand).astype(jnp.int32), axis=1, keepdims=True)
        tu = jnp.where(cnt >= _K, cand, tu)
    thr = tu ^ _MINT

    gti = (k2 > thr).astype(jnp.int32)
    eqi = (k2 == thr).astype(jnp.int32)
    g = jnp.sum(gti, axis=1, keepdims=True)
    need = _K - g

    incl = _cumsum_lanes(eqi)
    tot = jnp.sum(eqi, axis=1, keepdims=True)
    pos_l = incl - eqi
    pos_r = tot - incl
    rowi = jax.lax.broadcasted_iota(jnp.int32, (2 * _B, _N), 0)
    tie_pos = jnp.where(rowi < _B, pos_l, pos_r)
    sel = gti + eqi * (tie_pos < need).astype(jnp.int32)

    slot = _cumsum_lanes(sel) - sel
    out_ref[...] = jnp.where(sel > 0, slot, -1)


def _loss_kernel(s_ref, p_ref, yt_ref, w_ref, m_ref, pt_ref, pb_ref,
                 out_ref, acc_ref):
    b = pl.program_id(0)

    @pl.when(b == 0)
    def _init():
        for i in range(6):
            acc_ref[i] = 0.0

    r256 = jax.lax.broadcasted_iota(jnp.int32, (_KPAD, _N), 0)
    dnum_c = (((1,), (1,)), ((), ()))
    rank_part = 0.0
    for t in range(_BPS):
        sw = jnp.concatenate([s_ref[t], w_ref[t]], axis=0)
        p_top = (pt_ref[t] == r256).astype(jnp.float32)
        p_bot = (pb_ref[t] == r256).astype(jnp.float32)
        top_g = jax.lax.dot_general(p_top, sw, dnum_c,
                                    preferred_element_type=jnp.float32)
        bot_g = jax.lax.dot_general(sw, p_bot, dnum_c,
                                    preferred_element_type=jnp.float32)
        st = top_g[:, 0:1]
        at = jnp.sqrt(top_g[:, 1:2])
        sb = bot_g[0:1, :]
        ab = jnp.sqrt(bot_g[1:2, :])

        d = sb - st
        sp = jnp.maximum(d, 0.0) + jnp.log1p(jnp.exp(-jnp.abs(d)))
        wp = at * ab
        num = jnp.sum(sp * wp)
        den = jnp.sum(at) * jnp.sum(ab)
        rank_part += num / (den + 1e-8)

    w = w_ref[:, 0, :]
    p = p_ref[:, 0, :]
    ytr = yt_ref[:, 0, :]
    m = m_ref[:, 0, :]
    logp = jnp.maximum(jnp.log(p), -100.0)
    log1mp = jnp.maximum(jnp.log(1.0 - p), -100.0)
    bce = -(ytr * logp + (1.0 - ytr) * log1mp)
    mw = w * m
    t_den = jnp.sum(mw, axis=1, keepdims=True)
    t_num = jnp.sum(bce * mw, axis=1, keepdims=True)
    validf = (t_den > 0.0).astype(jnp.float32)
    pb_trade = t_num / (t_den + 1e-8)

    acc_ref[0] += rank_part
    acc_ref[1] += jnp.sum(validf * pb_trade)
    acc_ref[2] += jnp.sum(validf)
    acc_ref[3] += jnp.sum(p * m)
    acc_ref[4] += jnp.sum(m)

    @pl.when(b == _B // _BPS - 1)
    def _finish():
        avg_rank = acc_ref[0] / float(_B)
        avg_trade = acc_ref[1] / jnp.maximum(acc_ref[2], 1.0)
        out_ref[0] = avg_rank + _TRADE_LAMBDA * avg_trade
        out_ref[1] = avg_rank
        out_ref[2] = avg_trade
        out_ref[3] = acc_ref[3] / jnp.maximum(acc_ref[4], 1.0)


def kernel(scores, p_trade, y_rank, y_trade, weights, mask):
    slots = pl.pallas_call(
        _select_kernel,
        out_shape=jax.ShapeDtypeStruct((2 * _B, _N), jnp.int32),
    )(y_rank)
    ptop = slots[:_B].reshape(_B, 1, _N)
    pbot = slots[_B:].reshape(_B, 1, _N)

    blk = pl.BlockSpec((_BPS, 1, _N), lambda b: (b, 0, 0))
    args = (scores, p_trade, y_trade, weights, mask.astype(jnp.float32))
    args = tuple(a.reshape(_B, 1, _N) for a in args) + (ptop, pbot)
    out = pl.pallas_call(
        _loss_kernel,
        grid=(_B // _BPS,),
        in_specs=[blk] * 7,
        out_specs=pl.BlockSpec(memory_space=pltpu.SMEM),
        out_shape=jax.ShapeDtypeStruct((4,), jnp.float32),
        scratch_shapes=[pltpu.SMEM((6,), jnp.float32)],
    )(*args)
    return (out[0], out[1], out[2], out[3])


# speedup vs baseline: 803.1331x; 1.0666x over previous
import jax
import jax.numpy as jnp
from jax.experimental import pallas as pl
from jax.experimental.pallas import tpu as pltpu

_TRADE_LAMBDA = 0.25
_B = 64
_N = 1024
_K = 204
_KPAD = 256
_MINT = -(1 << 31)
_BPS = 8


def _cumsum_lanes(x):
    r, c = x.shape
    sh = 1
    while sh < c:
        z = jnp.zeros((r, sh), x.dtype)
        x = x + jnp.concatenate([z, x[:, :c - sh]], axis=1)
        sh *= 2
    return x


def _select_kernel(y_ref, out_ref):
    y = y_ref[...]
    y = jnp.where(y == 0.0, 0.0, y)
    bits = jax.lax.bitcast_convert_type(y, jnp.int32)
    key = jnp.where(bits >= 0, bits, ~(bits & 0x7FFFFFFF))
    k2 = jnp.concatenate([key, ~key], axis=0)

    tu = jnp.zeros((2 * _B, 1), jnp.int32)
    for bit in range(31, -1, -1):
        bv = jnp.asarray(_MINT if bit == 31 else (1 << bit), jnp.int32)
        cand = tu | bv
        scand = cand ^ _MINT
        cnt = jnp.sum((k2 >= scand).astype(jnp.int32), axis=1, keepdims=True)
        tu = jnp.where(cnt >= _K, cand, tu)
    thr = tu ^ _MINT

    gti = (k2 > thr).astype(jnp.int32)
    eqi = (k2 == thr).astype(jnp.int32)
    g = jnp.sum(gti, axis=1, keepdims=True)
    need = _K - g

    incl = _cumsum_lanes(eqi)
    tot = jnp.sum(eqi, axis=1, keepdims=True)
    pos_l = incl - eqi
    pos_r = tot - incl
    rowi = jax.lax.broadcasted_iota(jnp.int32, (2 * _B, _N), 0)
    tie_pos = jnp.where(rowi < _B, pos_l, pos_r)
    sel = gti + eqi * (tie_pos < need).astype(jnp.int32)

    slot = _cumsum_lanes(sel) - sel
    out_ref[...] = jnp.where(sel > 0, slot, -1)


def _loss_kernel(s_ref, p_ref, yt_ref, w_ref, m_ref, pt_ref, pb_ref,
                 out_ref, acc_ref):
    b = pl.program_id(0)

    @pl.when(b == 0)
    def _init():
        for i in range(6):
            acc_ref[i] = 0.0

    r256 = jax.lax.broadcasted_iota(jnp.int32, (_KPAD, _N), 0)
    dnum_c = (((1,), (1,)), ((), ()))
    rank_part = 0.0
    for t in range(_BPS):
        sw = jnp.concatenate([s_ref[t], w_ref[t]], axis=0)
        p_top = (pt_ref[t] == r256).astype(jnp.float32)
        p_bot = (pb_ref[t] == r256).astype(jnp.float32)
        top_g = jax.lax.dot_general(p_top, sw, dnum_c,
                                    preferred_element_type=jnp.float32)
        bot_g = jax.lax.dot_general(sw, p_bot, dnum_c,
                                    preferred_element_type=jnp.float32)
        st = top_g[:, 0:1]
        at = jnp.sqrt(top_g[:, 1:2])
        sb = bot_g[0:1, :]
        ab = jnp.sqrt(bot_g[1:2, :])

        d = sb - st
        sp = jnp.maximum(d, 0.0) + jnp.log1p(jnp.exp(-jnp.abs(d)))
        wp = at * ab
        num = jnp.sum(sp * wp)
        den = jnp.sum(at) * jnp.sum(ab)
        rank_part += num / (den + 1e-8)

    w = w_ref[:, 0, :]
    p = p_ref[:, 0, :]
    ytr = yt_ref[:, 0, :]
    m = m_ref[:, 0, :]
    logp = jnp.maximum(jnp.log(p), -100.0)
    log1mp = jnp.maximum(jnp.log(1.0 - p), -100.0)
    bce = -(ytr * logp + (1.0 - ytr) * log1mp)
    mw = w * m
    t_den = jnp.sum(mw, axis=1, keepdims=True)
    t_num = jnp.sum(bce * mw, axis=1, keepdims=True)
    validf = (t_den > 0.0).astype(jnp.float32)
    pb_trade = t_num / (t_den + 1e-8)

    acc_ref[0] += rank_part
    acc_ref[1] += jnp.sum(validf * pb_trade)
    acc_ref[2] += jnp.sum(validf)
    acc_ref[3] += jnp.sum(p * m)
    acc_ref[4] += jnp.sum(m)

    @pl.when(b == _B // _BPS - 1)
    def _finish():
        avg_rank = acc_ref[0] / float(_B)
        avg_trade = acc_ref[1] / jnp.maximum(acc_ref[2], 1.0)
        out_ref[0] = avg_rank + _TRADE_LAMBDA * avg_trade
        out_ref[1] = avg_rank
        out_ref[2] = avg_trade
        out_ref[3] = acc_ref[3] / jnp.maximum(acc_ref[4], 1.0)


def kernel(scores, p_trade, y_rank, y_trade, weights, mask):
    slots = pl.pallas_call(
        _select_kernel,
        out_shape=jax.ShapeDtypeStruct((2 * _B, _N), jnp.int32),
    )(y_rank)
    ptop = slots[:_B].reshape(_B, 1, _N)
    pbot = slots[_B:].reshape(_B, 1, _N)

    blk = pl.BlockSpec((_BPS, 1, _N), lambda b: (b, 0, 0))
    args = (scores, p_trade, y_trade, weights, mask.astype(jnp.float32))
    args = tuple(a.reshape(_B, 1, _N) for a in args) + (ptop, pbot)
    out = pl.pallas_call(
        _loss_kernel,
        grid=(_B // _BPS,),
        in_specs=[blk] * 7,
        out_specs=pl.BlockSpec(memory_space=pltpu.SMEM),
        out_shape=jax.ShapeDtypeStruct((4,), jnp.float32),
        scratch_shapes=[pltpu.SMEM((6,), jnp.float32)],
    )(*args)
    return (out[0], out[1], out[2], out[3])


# speedup vs baseline: 865.1027x; 1.0772x over previous
import jax
import jax.numpy as jnp
from jax.experimental import pallas as pl
from jax.experimental.pallas import tpu as pltpu

_TRADE_LAMBDA = 0.25
_B = 64
_N = 1024
_K = 204
_KPAD = 256
_MINT = -(1 << 31)
_BPS = 16


def _cumsum_lanes(x):
    r, c = x.shape
    sh = 1
    while sh < c:
        z = jnp.zeros((r, sh), x.dtype)
        x = x + jnp.concatenate([z, x[:, :c - sh]], axis=1)
        sh *= 2
    return x


def _select_kernel(y_ref, out_ref):
    y = y_ref[...]
    y = jnp.where(y == 0.0, 0.0, y)
    bits = jax.lax.bitcast_convert_type(y, jnp.int32)
    key = jnp.where(bits >= 0, bits, ~(bits & 0x7FFFFFFF))
    k2 = jnp.concatenate([key, ~key], axis=0)

    tu = jnp.zeros((2 * _B, 1), jnp.int32)
    for bit in range(31, -1, -1):
        bv = jnp.asarray(_MINT if bit == 31 else (1 << bit), jnp.int32)
        cand = tu | bv
        scand = cand ^ _MINT
        cnt = jnp.sum((k2 >= scand).astype(jnp.int32), axis=1, keepdims=True)
        tu = jnp.where(cnt >= _K, cand, tu)
    thr = tu ^ _MINT

    gti = (k2 > thr).astype(jnp.int32)
    eqi = (k2 == thr).astype(jnp.int32)
    g = jnp.sum(gti, axis=1, keepdims=True)
    need = _K - g

    incl = _cumsum_lanes(eqi)
    tot = jnp.sum(eqi, axis=1, keepdims=True)
    pos_l = incl - eqi
    pos_r = tot - incl
    rowi = jax.lax.broadcasted_iota(jnp.int32, (2 * _B, _N), 0)
    tie_pos = jnp.where(rowi < _B, pos_l, pos_r)
    sel = gti + eqi * (tie_pos < need).astype(jnp.int32)

    slot = _cumsum_lanes(sel) - sel
    out_ref[...] = jnp.where(sel > 0, slot, -1)


def _loss_kernel(s_ref, p_ref, yt_ref, w_ref, m_ref, pt_ref, pb_ref,
                 out_ref, acc_ref):
    b = pl.program_id(0)

    @pl.when(b == 0)
    def _init():
        for i in range(6):
            acc_ref[i] = 0.0

    r256 = jax.lax.broadcasted_iota(jnp.int32, (_KPAD, _N), 0)
    dnum_c = (((1,), (1,)), ((), ()))
    rank_part = 0.0
    for t in range(_BPS):
        sw = jnp.concatenate([s_ref[t], w_ref[t]], axis=0)
        p_top = (pt_ref[t] == r256).astype(jnp.float32)
        p_bot = (pb_ref[t] == r256).astype(jnp.float32)
        top_g = jax.lax.dot_general(p_top, sw, dnum_c,
                                    preferred_element_type=jnp.float32)
        bot_g = jax.lax.dot_general(sw, p_bot, dnum_c,
                                    preferred_element_type=jnp.float32)
        st = top_g[:, 0:1]
        at = jnp.sqrt(top_g[:, 1:2])
        sb = bot_g[0:1, :]
        ab = jnp.sqrt(bot_g[1:2, :])

        d = jnp.minimum(sb - st, 60.0)
        sp = jnp.log1p(jnp.exp(d))
        wp = at * ab
        num = jnp.sum(sp * wp)
        den = jnp.sum(at) * jnp.sum(ab)
        rank_part += num / (den + 1e-8)

    w = w_ref[:, 0, :]
    p = p_ref[:, 0, :]
    ytr = yt_ref[:, 0, :]
    m = m_ref[:, 0, :]
    logp = jnp.maximum(jnp.log(p), -100.0)
    log1mp = jnp.maximum(jnp.log(1.0 - p), -100.0)
    bce = -(ytr * logp + (1.0 - ytr) * log1mp)
    mw = w * m
    t_den = jnp.sum(mw, axis=1, keepdims=True)
    t_num = jnp.sum(bce * mw, axis=1, keepdims=True)
    validf = (t_den > 0.0).astype(jnp.float32)
    pb_trade = t_num / (t_den + 1e-8)

    acc_ref[0] += rank_part
    acc_ref[1] += jnp.sum(validf * pb_trade)
    acc_ref[2] += jnp.sum(validf)
    acc_ref[3] += jnp.sum(p * m)
    acc_ref[4] += jnp.sum(m)

    @pl.when(b == _B // _BPS - 1)
    def _finish():
        avg_rank = acc_ref[0] / float(_B)
        avg_trade = acc_ref[1] / jnp.maximum(acc_ref[2], 1.0)
        out_ref[0] = avg_rank + _TRADE_LAMBDA * avg_trade
        out_ref[1] = avg_rank
        out_ref[2] = avg_trade
        out_ref[3] = acc_ref[3] / jnp.maximum(acc_ref[4], 1.0)


def kernel(scores, p_trade, y_rank, y_trade, weights, mask):
    slots = pl.pallas_call(
        _select_kernel,
        out_shape=jax.ShapeDtypeStruct((2 * _B, _N), jnp.int32),
    )(y_rank)
    ptop = slots[:_B].reshape(_B, 1, _N)
    pbot = slots[_B:].reshape(_B, 1, _N)

    blk = pl.BlockSpec((_BPS, 1, _N), lambda b: (b, 0, 0))
    args = (scores, p_trade, y_trade, weights, mask.astype(jnp.float32))
    args = tuple(a.reshape(_B, 1, _N) for a in args) + (ptop, pbot)
    out = pl.pallas_call(
        _loss_kernel,
        grid=(_B // _BPS,),
        in_specs=[blk] * 7,
        out_specs=pl.BlockSpec(memory_space=pltpu.SMEM),
        out_shape=jax.ShapeDtypeStruct((4,), jnp.float32),
        scratch_shapes=[pltpu.SMEM((6,), jnp.float32)],
    )(*args)
    return (out[0], out[1], out[2], out[3])


# speedup vs baseline: 946.8553x; 1.0945x over previous
import jax
import jax.numpy as jnp
from jax.experimental import pallas as pl
from jax.experimental.pallas import tpu as pltpu

_TRADE_LAMBDA = 0.25
_B = 64
_N = 1024
_K = 204
_KPAD = 256
_MINT = -(1 << 31)
_BPS = 16


def _cumsum_lanes(x):
    r, c = x.shape
    sh = 1
    while sh < c:
        z = jnp.zeros((r, sh), x.dtype)
        x = x + jnp.concatenate([z, x[:, :c - sh]], axis=1)
        sh *= 2
    return x


def _select(y):
    y = jnp.where(y == 0.0, 0.0, y)
    bits = jax.lax.bitcast_convert_type(y, jnp.int32)
    key = jnp.where(bits >= 0, bits, ~(bits & 0x7FFFFFFF))
    k2 = jnp.concatenate([key, ~key], axis=0)

    tu = jnp.zeros((2 * _B, 1), jnp.int32)
    for bit in range(31, -1, -1):
        bv = jnp.asarray(_MINT if bit == 31 else (1 << bit), jnp.int32)
        cand = tu | bv
        scand = cand ^ _MINT
        cnt = jnp.sum((k2 >= scand).astype(jnp.int32), axis=1, keepdims=True)
        tu = jnp.where(cnt >= _K, cand, tu)
    thr = tu ^ _MINT

    gti = (k2 > thr).astype(jnp.int32)
    eqi = (k2 == thr).astype(jnp.int32)
    g = jnp.sum(gti, axis=1, keepdims=True)
    need = _K - g

    incl = _cumsum_lanes(eqi)
    tot = jnp.sum(eqi, axis=1, keepdims=True)
    pos_l = incl - eqi
    pos_r = tot - incl
    rowi = jax.lax.broadcasted_iota(jnp.int32, (2 * _B, _N), 0)
    tie_pos = jnp.where(rowi < _B, pos_l, pos_r)
    sel = gti + eqi * (tie_pos < need).astype(jnp.int32)

    slot = _cumsum_lanes(sel) - sel
    return jnp.where(sel > 0, slot, -1)


def _fused_kernel(y_ref, s_ref, p_ref, yt_ref, w_ref, m_ref,
                  out_ref, slots_ref, acc_ref):
    i = pl.program_id(0)

    @pl.when(i == 0)
    def _select_step():
        for j in range(6):
            acc_ref[j] = 0.0
        slots_ref[...] = _select(y_ref[...])

    @pl.when(i > 0)
    def _loss_step():
        base = pl.multiple_of((i - 1) * _BPS, _BPS)
        pt_blk = slots_ref[pl.ds(base, _BPS), :]
        pb_blk = slots_ref[pl.ds(_B + base, _BPS), :]

        r256 = jax.lax.broadcasted_iota(jnp.int32, (_KPAD, _N), 0)
        dnum_c = (((1,), (1,)), ((), ()))
        rank_part = 0.0
        for t in range(_BPS):
            sw = jnp.concatenate([s_ref[t], w_ref[t]], axis=0)
            p_top = (pt_blk[t:t + 1, :] == r256).astype(jnp.float32)
            p_bot = (pb_blk[t:t + 1, :] == r256).astype(jnp.float32)
            top_g = jax.lax.dot_general(p_top, sw, dnum_c,
                                        preferred_element_type=jnp.float32)
            bot_g = jax.lax.dot_general(sw, p_bot, dnum_c,
                                        preferred_element_type=jnp.float32)
            st = top_g[:, 0:1]
            at = jnp.sqrt(top_g[:, 1:2])
            sb = bot_g[0:1, :]
            ab = jnp.sqrt(bot_g[1:2, :])

            d = jnp.minimum(sb - st, 60.0)
            sp = jnp.log1p(jnp.exp(d))
            wp = at * ab
            num = jnp.sum(sp * wp)
            den = jnp.sum(at) * jnp.sum(ab)
            rank_part += num / (den + 1e-8)

        w = w_ref[:, 0, :]
        p = p_ref[:, 0, :]
        ytr = yt_ref[:, 0, :]
        m = m_ref[:, 0, :]
        logp = jnp.maximum(jnp.log(p), -100.0)
        log1mp = jnp.maximum(jnp.log(1.0 - p), -100.0)
        bce = -(ytr * logp + (1.0 - ytr) * log1mp)
        mw = w * m
        t_den = jnp.sum(mw, axis=1, keepdims=True)
        t_num = jnp.sum(bce * mw, axis=1, keepdims=True)
        validf = (t_den > 0.0).astype(jnp.float32)
        pb_trade = t_num / (t_den + 1e-8)

        acc_ref[0] += rank_part
        acc_ref[1] += jnp.sum(validf * pb_trade)
        acc_ref[2] += jnp.sum(validf)
        acc_ref[3] += jnp.sum(p * m)
        acc_ref[4] += jnp.sum(m)

    @pl.when(i == _B // _BPS)
    def _finish():
        avg_rank = acc_ref[0] / float(_B)
        avg_trade = acc_ref[1] / jnp.maximum(acc_ref[2], 1.0)
        out_ref[0] = avg_rank + _TRADE_LAMBDA * avg_trade
        out_ref[1] = avg_rank
        out_ref[2] = avg_trade
        out_ref[3] = acc_ref[3] / jnp.maximum(acc_ref[4], 1.0)


def kernel(scores, p_trade, y_rank, y_trade, weights, mask):
    yspec = pl.BlockSpec((_B, _N), lambda i: (0, 0))
    blk = pl.BlockSpec((_BPS, 1, _N),
                       lambda i: (jnp.maximum(i - 1, 0), 0, 0))
    args = (scores, p_trade, y_trade, weights, mask.astype(jnp.float32))
    args = (y_rank,) + tuple(a.reshape(_B, 1, _N) for a in args)
    out = pl.pallas_call(
        _fused_kernel,
        grid=(1 + _B // _BPS,),
        in_specs=[yspec] + [blk] * 5,
        out_specs=pl.BlockSpec(memory_space=pltpu.SMEM),
        out_shape=jax.ShapeDtypeStruct((4,), jnp.float32),
        scratch_shapes=[pltpu.VMEM((2 * _B, _N), jnp.int32),
                        pltpu.SMEM((6,), jnp.float32)],
    )(*args)
    return (out[0], out[1], out[2], out[3])


# speedup vs baseline: 963.7714x; 1.0179x over previous
import jax
import jax.numpy as jnp
from jax.experimental import pallas as pl
from jax.experimental.pallas import tpu as pltpu

_TRADE_LAMBDA = 0.25
_B = 64
_N = 1024
_K = 204
_KPAD = 256
_MINT = -(1 << 31)
_BPS = 32


def _cumsum_lanes(x):
    r, c = x.shape
    sh = 1
    while sh < c:
        z = jnp.zeros((r, sh), x.dtype)
        x = x + jnp.concatenate([z, x[:, :c - sh]], axis=1)
        sh *= 2
    return x


def _select(y):
    y = jnp.where(y == 0.0, 0.0, y)
    bits = jax.lax.bitcast_convert_type(y, jnp.int32)
    key = jnp.where(bits >= 0, bits, ~(bits & 0x7FFFFFFF))
    k2 = jnp.concatenate([key, ~key], axis=0)

    tu = jnp.zeros((2 * _B, 1), jnp.int32)
    for bit in range(31, -1, -1):
        bv = jnp.asarray(_MINT if bit == 31 else (1 << bit), jnp.int32)
        cand = tu | bv
        scand = cand ^ _MINT
        cnt = jnp.sum((k2 >= scand).astype(jnp.int32), axis=1, keepdims=True)
        tu = jnp.where(cnt >= _K, cand, tu)
    thr = tu ^ _MINT

    gti = (k2 > thr).astype(jnp.int32)
    eqi = (k2 == thr).astype(jnp.int32)
    g = jnp.sum(gti, axis=1, keepdims=True)
    need = _K - g

    incl = _cumsum_lanes(eqi)
    tot = jnp.sum(eqi, axis=1, keepdims=True)
    pos_l = incl - eqi
    pos_r = tot - incl
    rowi = jax.lax.broadcasted_iota(jnp.int32, (2 * _B, _N), 0)
    tie_pos = jnp.where(rowi < _B, pos_l, pos_r)
    sel = gti + eqi * (tie_pos < need).astype(jnp.int32)

    slot = _cumsum_lanes(sel) - sel
    return jnp.where(sel > 0, slot, -1)


def _fused_kernel(y_ref, s_ref, p_ref, yt_ref, w_ref, m_ref,
                  out_ref, slots_ref, acc_ref):
    i = pl.program_id(0)

    @pl.when(i == 0)
    def _select_step():
        for j in range(6):
            acc_ref[j] = 0.0
        slots_ref[...] = _select(y_ref[...])

    @pl.when(i > 0)
    def _loss_step():
        base = pl.multiple_of((i - 1) * _BPS, _BPS)
        pt_blk = slots_ref[pl.ds(base, _BPS), :]
        pb_blk = slots_ref[pl.ds(_B + base, _BPS), :]

        r256 = jax.lax.broadcasted_iota(jnp.int32, (_KPAD, _N), 0)
        dnum_c = (((1,), (1,)), ((), ()))
        rank_part = 0.0
        for t in range(_BPS):
            sw = jnp.concatenate([s_ref[t], w_ref[t]], axis=0)
            p_top = (pt_blk[t:t + 1, :] == r256).astype(jnp.float32)
            p_bot = (pb_blk[t:t + 1, :] == r256).astype(jnp.float32)
            top_g = jax.lax.dot_general(p_top, sw, dnum_c,
                                        preferred_element_type=jnp.float32)
            bot_g = jax.lax.dot_general(sw, p_bot, dnum_c,
                                        preferred_element_type=jnp.float32)
            st = top_g[:, 0:1]
            at = jnp.sqrt(top_g[:, 1:2])
            sb = bot_g[0:1, :]
            ab = jnp.sqrt(bot_g[1:2, :])

            d = jnp.minimum(sb - st, 60.0)
            sp = jnp.log1p(jnp.exp(d))
            wp = at * ab
            num = jnp.sum(sp * wp)
            den = jnp.sum(at) * jnp.sum(ab)
            rank_part += num / (den + 1e-8)

        w = w_ref[:, 0, :]
        p = p_ref[:, 0, :]
        ytr = yt_ref[:, 0, :]
        m = m_ref[:, 0, :]
        logp = jnp.maximum(jnp.log(p), -100.0)
        log1mp = jnp.maximum(jnp.log(1.0 - p), -100.0)
        bce = -(ytr * logp + (1.0 - ytr) * log1mp)
        mw = w * m
        t_den = jnp.sum(mw, axis=1, keepdims=True)
        t_num = jnp.sum(bce * mw, axis=1, keepdims=True)
        validf = (t_den > 0.0).astype(jnp.float32)
        pb_trade = t_num / (t_den + 1e-8)

        acc_ref[0] += rank_part
        acc_ref[1] += jnp.sum(validf * pb_trade)
        acc_ref[2] += jnp.sum(validf)
        acc_ref[3] += jnp.sum(p * m)
        acc_ref[4] += jnp.sum(m)

    @pl.when(i == _B // _BPS)
    def _finish():
        avg_rank = acc_ref[0] / float(_B)
        avg_trade = acc_ref[1] / jnp.maximum(acc_ref[2], 1.0)
        out_ref[0] = avg_rank + _TRADE_LAMBDA * avg_trade
        out_ref[1] = avg_rank
        out_ref[2] = avg_trade
        out_ref[3] = acc_ref[3] / jnp.maximum(acc_ref[4], 1.0)


def kernel(scores, p_trade, y_rank, y_trade, weights, mask):
    yspec = pl.BlockSpec((_B, _N), lambda i: (0, 0))
    blk = pl.BlockSpec((_BPS, 1, _N),
                       lambda i: (jnp.maximum(i - 1, 0), 0, 0))
    args = (scores, p_trade, y_trade, weights, mask.astype(jnp.float32))
    args = (y_rank,) + tuple(a.reshape(_B, 1, _N) for a in args)
    out = pl.pallas_call(
        _fused_kernel,
        grid=(1 + _B // _BPS,),
        in_specs=[yspec] + [blk] * 5,
        out_specs=pl.BlockSpec(memory_space=pltpu.SMEM),
        out_shape=jax.ShapeDtypeStruct((4,), jnp.float32),
        scratch_shapes=[pltpu.VMEM((2 * _B, _N), jnp.int32),
                        pltpu.SMEM((6,), jnp.float32)],
    )(*args)
    return (out[0], out[1], out[2], out[3])


# speedup vs baseline: 1145.3054x; 1.1884x over previous
import jax
import jax.numpy as jnp
from jax.experimental import pallas as pl
from jax.experimental.pallas import tpu as pltpu

_TRADE_LAMBDA = 0.25
_B = 64
_N = 1024
_K = 204
_KPAD = 256
_MINT = -(1 << 31)
_BPS = 32


def _cumsum_lanes(x):
    r, c = x.shape
    sh = 1
    while sh < c:
        z = jnp.zeros((r, sh), x.dtype)
        x = x + jnp.concatenate([z, x[:, :c - sh]], axis=1)
        sh *= 2
    return x


def _select(y):
    y = jnp.where(y == 0.0, 0.0, y)
    bits = jax.lax.bitcast_convert_type(y, jnp.int32)
    key = jnp.where(bits >= 0, bits, ~(bits & 0x7FFFFFFF))
    k2 = jnp.concatenate([key, ~key], axis=0)

    tu = jnp.zeros((2 * _B, 1), jnp.int32)
    for bit in range(31, -1, -1):
        bv = jnp.asarray(_MINT if bit == 31 else (1 << bit), jnp.int32)
        cand = tu | bv
        scand = cand ^ _MINT
        cnt = jnp.sum((k2 >= scand).astype(jnp.int32), axis=1, keepdims=True)
        tu = jnp.where(cnt >= _K, cand, tu)
    thr = tu ^ _MINT

    gti = (k2 > thr).astype(jnp.int32)
    eqi = (k2 == thr).astype(jnp.int32)
    g = jnp.sum(gti, axis=1, keepdims=True)
    need = _K - g

    incl = _cumsum_lanes(eqi)
    tot = jnp.sum(eqi, axis=1, keepdims=True)
    pos_l = incl - eqi
    pos_r = tot - incl
    rowi = jax.lax.broadcasted_iota(jnp.int32, (2 * _B, _N), 0)
    tie_pos = jnp.where(rowi < _B, pos_l, pos_r)
    sel = gti + eqi * (tie_pos < need).astype(jnp.int32)

    slot = _cumsum_lanes(sel) - sel
    return jnp.where(sel > 0, slot, -1)


def _fused_kernel(y_ref, s_ref, p_ref, yt_ref, w_ref, m_ref,
                  out_ref, slots_ref, acc_ref):
    i = pl.program_id(0)

    @pl.when(i == 0)
    def _select_step():
        for j in range(6):
            acc_ref[j] = 0.0
        slots_ref[...] = _select(y_ref[...])

    @pl.when(i > 0)
    def _loss_step():
        base = pl.multiple_of((i - 1) * _BPS, _BPS)
        pt_blk = slots_ref[pl.ds(base, _BPS), :]
        pb_blk = slots_ref[pl.ds(_B + base, _BPS), :]

        r256 = jax.lax.broadcasted_iota(jnp.int32, (_KPAD, _N), 0)
        dnum_c = (((1,), (1,)), ((), ()))
        rank_part = 0.0
        for t in range(_BPS):
            sw = jnp.concatenate([s_ref[t:t + 1, :], w_ref[t:t + 1, :]],
                                 axis=0)
            p_top = (pt_blk[t:t + 1, :] == r256).astype(jnp.float32)
            p_bot = (pb_blk[t:t + 1, :] == r256).astype(jnp.float32)
            top_g = jax.lax.dot_general(p_top, sw, dnum_c,
                                        preferred_element_type=jnp.float32)
            bot_g = jax.lax.dot_general(sw, p_bot, dnum_c,
                                        preferred_element_type=jnp.float32)
            st = top_g[:, 0:1]
            at = jnp.sqrt(top_g[:, 1:2])
            sb = bot_g[0:1, :]
            ab = jnp.sqrt(bot_g[1:2, :])

            d = jnp.minimum(sb - st, 60.0)
            sp = jnp.log1p(jnp.exp(d))
            wp = at * ab
            num = jnp.sum(sp * wp)
            den = jnp.sum(at) * jnp.sum(ab)
            rank_part += num / (den + 1e-8)

        w = w_ref[...]
        p = p_ref[...]
        ytr = yt_ref[...]
        m = m_ref[...]
        logp = jnp.maximum(jnp.log(p), -100.0)
        log1mp = jnp.maximum(jnp.log(1.0 - p), -100.0)
        bce = -(ytr * logp + (1.0 - ytr) * log1mp)
        mw = w * m
        t_den = jnp.sum(mw, axis=1, keepdims=True)
        t_num = jnp.sum(bce * mw, axis=1, keepdims=True)
        validf = (t_den > 0.0).astype(jnp.float32)
        pb_trade = t_num / (t_den + 1e-8)

        acc_ref[0] += rank_part
        acc_ref[1] += jnp.sum(validf * pb_trade)
        acc_ref[2] += jnp.sum(validf)
        acc_ref[3] += jnp.sum(p * m)
        acc_ref[4] += jnp.sum(m)

    @pl.when(i == _B // _BPS)
    def _finish():
        avg_rank = acc_ref[0] / float(_B)
        avg_trade = acc_ref[1] / jnp.maximum(acc_ref[2], 1.0)
        out_ref[0] = avg_rank + _TRADE_LAMBDA * avg_trade
        out_ref[1] = avg_rank
        out_ref[2] = avg_trade
        out_ref[3] = acc_ref[3] / jnp.maximum(acc_ref[4], 1.0)


def kernel(scores, p_trade, y_rank, y_trade, weights, mask):
    yspec = pl.BlockSpec((_B, _N), lambda i: (0, 0))
    blk = pl.BlockSpec((_BPS, _N), lambda i: (jnp.maximum(i - 1, 0), 0))
    args = (y_rank, scores, p_trade, y_trade, weights,
            mask.astype(jnp.float32))
    out = pl.pallas_call(
        _fused_kernel,
        grid=(1 + _B // _BPS,),
        in_specs=[yspec] + [blk] * 5,
        out_specs=pl.BlockSpec(memory_space=pltpu.SMEM),
        out_shape=jax.ShapeDtypeStruct((4,), jnp.float32),
        scratch_shapes=[pltpu.VMEM((2 * _B, _N), jnp.int32),
                        pltpu.SMEM((6,), jnp.float32)],
    )(*args)
    return (out[0], out[1], out[2], out[3])


# speedup vs baseline: 1332.9986x; 1.1639x over previous
import jax
import jax.numpy as jnp
from jax.experimental import pallas as pl
from jax.experimental.pallas import tpu as pltpu

_TRADE_LAMBDA = 0.25
_B = 64
_N = 1024
_K = 204
_KPAD = 208
_MINT = -(1 << 31)
_BPS = 32


def _cumsum_lanes(x):
    r, c = x.shape
    sh = 1
    while sh < c:
        z = jnp.zeros((r, sh), x.dtype)
        x = x + jnp.concatenate([z, x[:, :c - sh]], axis=1)
        sh *= 2
    return x


def _select(y):
    y = jnp.where(y == 0.0, 0.0, y)
    bits = jax.lax.bitcast_convert_type(y, jnp.int32)
    key = jnp.where(bits >= 0, bits, ~(bits & 0x7FFFFFFF))
    k2 = jnp.concatenate([key, ~key], axis=0)

    tu = jnp.zeros((2 * _B, 1), jnp.int32)
    for bit in range(31, -1, -1):
        bv = jnp.asarray(_MINT if bit == 31 else (1 << bit), jnp.int32)
        cand = tu | bv
        scand = cand ^ _MINT
        cnt = jnp.sum((k2 >= scand).astype(jnp.int32), axis=1, keepdims=True)
        tu = jnp.where(cnt >= _K, cand, tu)
    thr = tu ^ _MINT

    gti = (k2 > thr).astype(jnp.int32)
    eqi = (k2 == thr).astype(jnp.int32)
    g = jnp.sum(gti, axis=1, keepdims=True)
    need = _K - g

    incl = _cumsum_lanes(eqi)
    tot = jnp.sum(eqi, axis=1, keepdims=True)
    pos_l = incl - eqi
    pos_r = tot - incl
    rowi = jax.lax.broadcasted_iota(jnp.int32, (2 * _B, _N), 0)
    tie_pos = jnp.where(rowi < _B, pos_l, pos_r)
    sel = gti + eqi * (tie_pos < need).astype(jnp.int32)

    slot = _cumsum_lanes(sel) - sel
    return jnp.where(sel > 0, slot, -1)


def _fused_kernel(y_ref, s_ref, p_ref, yt_ref, w_ref, m_ref,
                  out_ref, slots_ref, acc_ref):
    i = pl.program_id(0)

    @pl.when(i == 0)
    def _select_step():
        for j in range(6):
            acc_ref[j] = 0.0
        slots_ref[...] = _select(y_ref[...])

    @pl.when(i > 0)
    def _loss_step():
        base = pl.multiple_of((i - 1) * _BPS, _BPS)
        pt_blk = slots_ref[pl.ds(base, _BPS), :]
        pb_blk = slots_ref[pl.ds(_B + base, _BPS), :]

        r256 = jax.lax.broadcasted_iota(jnp.int32, (_KPAD, _N), 0)
        dnum_c = (((1,), (1,)), ((), ()))
        rank_part = 0.0
        for t in range(_BPS):
            sw = jnp.concatenate([s_ref[t:t + 1, :], w_ref[t:t + 1, :]],
                                 axis=0)
            p_top = (pt_blk[t:t + 1, :] == r256).astype(jnp.float32)
            p_bot = (pb_blk[t:t + 1, :] == r256).astype(jnp.float32)
            top_g = jax.lax.dot_general(p_top, sw, dnum_c,
                                        preferred_element_type=jnp.float32)
            bot_g = jax.lax.dot_general(sw, p_bot, dnum_c,
                                        preferred_element_type=jnp.float32)
            st = top_g[:, 0:1]
            at = jnp.sqrt(top_g[:, 1:2])
            sb = bot_g[0:1, :]
            ab = jnp.sqrt(bot_g[1:2, :])

            d = jnp.minimum(sb - st, 60.0)
            sp = jnp.log1p(jnp.exp(d))
            wp = at * ab
            num = jnp.sum(sp * wp)
            den = jnp.sum(at) * jnp.sum(ab)
            rank_part += num / (den + 1e-8)

        w = w_ref[...]
        p = p_ref[...]
        ytr = yt_ref[...]
        m = m_ref[...]
        logp = jnp.maximum(jnp.log(p), -100.0)
        log1mp = jnp.maximum(jnp.log(1.0 - p), -100.0)
        bce = -(ytr * logp + (1.0 - ytr) * log1mp)
        mw = w * m
        t_den = jnp.sum(mw, axis=1, keepdims=True)
        t_num = jnp.sum(bce * mw, axis=1, keepdims=True)
        validf = (t_den > 0.0).astype(jnp.float32)
        pb_trade = t_num / (t_den + 1e-8)

        acc_ref[0] += rank_part
        acc_ref[1] += jnp.sum(validf * pb_trade)
        acc_ref[2] += jnp.sum(validf)
        acc_ref[3] += jnp.sum(p * m)
        acc_ref[4] += jnp.sum(m)

    @pl.when(i == _B // _BPS)
    def _finish():
        avg_rank = acc_ref[0] / float(_B)
        avg_trade = acc_ref[1] / jnp.maximum(acc_ref[2], 1.0)
        out_ref[0] = avg_rank + _TRADE_LAMBDA * avg_trade
        out_ref[1] = avg_rank
        out_ref[2] = avg_trade
        out_ref[3] = acc_ref[3] / jnp.maximum(acc_ref[4], 1.0)


def kernel(scores, p_trade, y_rank, y_trade, weights, mask):
    yspec = pl.BlockSpec((_B, _N), lambda i: (0, 0))
    blk = pl.BlockSpec((_BPS, _N), lambda i: (jnp.maximum(i - 1, 0), 0))
    args = (y_rank, scores, p_trade, y_trade, weights,
            mask.astype(jnp.float32))
    out = pl.pallas_call(
        _fused_kernel,
        grid=(1 + _B // _BPS,),
        in_specs=[yspec] + [blk] * 5,
        out_specs=pl.BlockSpec(memory_space=pltpu.SMEM),
        out_shape=jax.ShapeDtypeStruct((4,), jnp.float32),
        scratch_shapes=[pltpu.VMEM((2 * _B, _N), jnp.int32),
                        pltpu.SMEM((6,), jnp.float32)],
    )(*args)
    return (out[0], out[1], out[2], out[3])


# speedup vs baseline: 1341.0510x; 1.0060x over previous
import jax
import jax.numpy as jnp
from jax.experimental import pallas as pl
from jax.experimental.pallas import tpu as pltpu

_TRADE_LAMBDA = 0.25
_B = 64
_N = 1024
_K = 204
_KPAD = 208
_MINT = -(1 << 31)
_BPS = 64


def _cumsum_lanes(x):
    r, c = x.shape
    sh = 1
    while sh < c:
        z = jnp.zeros((r, sh), x.dtype)
        x = x + jnp.concatenate([z, x[:, :c - sh]], axis=1)
        sh *= 2
    return x


def _select(y):
    y = jnp.where(y == 0.0, 0.0, y)
    bits = jax.lax.bitcast_convert_type(y, jnp.int32)
    key = jnp.where(bits >= 0, bits, ~(bits & 0x7FFFFFFF))
    k2 = jnp.concatenate([key, ~key], axis=0)

    tu = jnp.zeros((2 * _B, 1), jnp.int32)
    for bit in range(31, -1, -1):
        bv = jnp.asarray(_MINT if bit == 31 else (1 << bit), jnp.int32)
        cand = tu | bv
        scand = cand ^ _MINT
        cnt = jnp.sum((k2 >= scand).astype(jnp.int32), axis=1, keepdims=True)
        tu = jnp.where(cnt >= _K, cand, tu)
    thr = tu ^ _MINT

    gti = (k2 > thr).astype(jnp.int32)
    eqi = (k2 == thr).astype(jnp.int32)
    g = jnp.sum(gti, axis=1, keepdims=True)
    need = _K - g

    incl = _cumsum_lanes(eqi)
    tot = jnp.sum(eqi, axis=1, keepdims=True)
    pos_l = incl - eqi
    pos_r = tot - incl
    rowi = jax.lax.broadcasted_iota(jnp.int32, (2 * _B, _N), 0)
    tie_pos = jnp.where(rowi < _B, pos_l, pos_r)
    sel = gti + eqi * (tie_pos < need).astype(jnp.int32)

    slot = _cumsum_lanes(sel) - sel
    return jnp.where(sel > 0, slot, -1)


def _fused_kernel(y_ref, s_ref, p_ref, yt_ref, w_ref, m_ref,
                  out_ref, slots_ref, acc_ref):
    i = pl.program_id(0)

    @pl.when(i == 0)
    def _select_step():
        for j in range(6):
            acc_ref[j] = 0.0
        slots_ref[...] = _select(y_ref[...])

    @pl.when(i > 0)
    def _loss_step():
        base = pl.multiple_of((i - 1) * _BPS, _BPS)
        pt_blk = slots_ref[pl.ds(base, _BPS), :]
        pb_blk = slots_ref[pl.ds(_B + base, _BPS), :]

        r256 = jax.lax.broadcasted_iota(jnp.int32, (_KPAD, _N), 0)
        dnum_c = (((1,), (1,)), ((), ()))
        rank_part = 0.0
        for t in range(_BPS):
            sw = jnp.concatenate([s_ref[t:t + 1, :], w_ref[t:t + 1, :]],
                                 axis=0)
            p_top = (pt_blk[t:t + 1, :] == r256).astype(jnp.float32)
            p_bot = (pb_blk[t:t + 1, :] == r256).astype(jnp.float32)
            top_g = jax.lax.dot_general(p_top, sw, dnum_c,
                                        preferred_element_type=jnp.float32)
            bot_g = jax.lax.dot_general(sw, p_bot, dnum_c,
                                        preferred_element_type=jnp.float32)
            st = top_g[:, 0:1]
            at = jnp.sqrt(top_g[:, 1:2])
            sb = bot_g[0:1, :]
            ab = jnp.sqrt(bot_g[1:2, :])

            d = jnp.minimum(sb - st, 60.0)
            sp = jnp.log1p(jnp.exp(d))
            wp = at * ab
            num = jnp.sum(sp * wp)
            den = jnp.sum(at) * jnp.sum(ab)
            rank_part += num / (den + 1e-8)

        w = w_ref[...]
        p = p_ref[...]
        ytr = yt_ref[...]
        m = m_ref[...]
        logp = jnp.maximum(jnp.log(p), -100.0)
        log1mp = jnp.maximum(jnp.log(1.0 - p), -100.0)
        bce = -(ytr * logp + (1.0 - ytr) * log1mp)
        mw = w * m
        t_den = jnp.sum(mw, axis=1, keepdims=True)
        t_num = jnp.sum(bce * mw, axis=1, keepdims=True)
        validf = (t_den > 0.0).astype(jnp.float32)
        pb_trade = t_num / (t_den + 1e-8)

        acc_ref[0] += rank_part
        acc_ref[1] += jnp.sum(validf * pb_trade)
        acc_ref[2] += jnp.sum(validf)
        acc_ref[3] += jnp.sum(p * m)
        acc_ref[4] += jnp.sum(m)

    @pl.when(i == _B // _BPS)
    def _finish():
        avg_rank = acc_ref[0] / float(_B)
        avg_trade = acc_ref[1] / jnp.maximum(acc_ref[2], 1.0)
        out_ref[0] = avg_rank + _TRADE_LAMBDA * avg_trade
        out_ref[1] = avg_rank
        out_ref[2] = avg_trade
        out_ref[3] = acc_ref[3] / jnp.maximum(acc_ref[4], 1.0)


def kernel(scores, p_trade, y_rank, y_trade, weights, mask):
    yspec = pl.BlockSpec((_B, _N), lambda i: (0, 0))
    blk = pl.BlockSpec((_BPS, _N), lambda i: (jnp.maximum(i - 1, 0), 0))
    args = (y_rank, scores, p_trade, y_trade, weights,
            mask.astype(jnp.float32))
    out = pl.pallas_call(
        _fused_kernel,
        grid=(1 + _B // _BPS,),
        in_specs=[yspec] + [blk] * 5,
        out_specs=pl.BlockSpec(memory_space=pltpu.SMEM),
        out_shape=jax.ShapeDtypeStruct((4,), jnp.float32),
        scratch_shapes=[pltpu.VMEM((2 * _B, _N), jnp.int32),
                        pltpu.SMEM((6,), jnp.float32)],
    )(*args)
    return (out[0], out[1], out[2], out[3])
